# baseline (speedup 1.0000x reference)
import functools

import jax
import jax.numpy as jnp
from jax import lax
from jax.experimental import pallas as pl
from jax.experimental.pallas import tpu as pltpu
from jax.experimental.pallas import tpu_sc as plsc

_TABLE = 100000
_D = 1024
_PRIMES = (17, 31, 53, 79)
_NC = 2
_NS = 16
_NW = _NC * _NS
_CHUNK = 16


_SUB = _CHUNK // 2


def _sc_gather_sum(ids, table, off, n_out):
    n_per_w = n_out // _NW
    n_chunks = n_per_w // _CHUNK
    mesh = plsc.VectorSubcoreMesh(
        core_axis_name="c", subcore_axis_name="s",
        num_cores=_NC, num_subcores=_NS,
    )

    @functools.partial(
        pl.kernel,
        out_type=jax.ShapeDtypeStruct((n_out, _D), jnp.float32),
        mesh=mesh,
        scratch_types=[
            pltpu.VMEM((n_per_w,), jnp.int32),
            pltpu.VMEM((4 * _CHUNK,), jnp.int32),
            pltpu.VMEM((4 * _SUB, _D), jnp.float32),
            pltpu.VMEM((4 * _SUB, _D), jnp.float32),
            pltpu.VMEM((_SUB, _D), jnp.float32),
            pltpu.VMEM((_SUB, _D), jnp.float32),
            pltpu.SemaphoreType.DMA,
            pltpu.SemaphoreType.DMA,
            pltpu.SemaphoreType.DMA,
            pltpu.SemaphoreType.DMA,
        ],
    )
    def kern(ids_hbm, table_hbm, out_hbm, ids_v, idx_v,
             buf0, buf1, acc0, acc1, sg0, sg1, so0, so1):
        wid = lax.axis_index("s") * _NC + lax.axis_index("c")
        base = wid * n_per_w
        pltpu.sync_copy(ids_hbm.at[pl.ds(off + base, n_per_w)], ids_v)

        def comp_idx(tc):
            ids16 = ids_v[pl.ds(tc * _CHUNK, _CHUNK)]
            for h in range(4):
                idx_v[pl.ds(h * _CHUNK, _CHUNK)] = (ids16 * _PRIMES[h]) % _TABLE

        def fire_gather(buf, sem, off):
            for h in range(4):
                pltpu.async_copy(
                    table_hbm.at[idx_v.at[pl.ds(h * _CHUNK + off, _SUB)]],
                    buf.at[pl.ds(h * _SUB, _SUB)], sem)

        def drain_gather(buf, sem):
            pltpu.make_async_copy(table_hbm.at[pl.ds(0, 4 * _SUB)], buf,
                                  sem).wait()

        def reduce_into(buf, acc):
            def rj(j, c):
                for u in range(2):
                    s = pl.ds(j * 32 + u * 16, 16)
                    for t in range(_SUB):
                        acc[t, s] = ((buf[t, s] + buf[_SUB + t, s])
                                     + (buf[2 * _SUB + t, s]
                                        + buf[3 * _SUB + t, s]))
                return c
            lax.fori_loop(0, _D // 32, rj, 0)

        comp_idx(0)
        fire_gather(buf0, sg0, 0)

        def body(tc, carry):
            row0 = base + tc * _CHUNK
            fire_gather(buf1, sg1, _SUB)

            drain_gather(buf0, sg0)

            @pl.when(tc != 0)
            def _():
                pltpu.make_async_copy(
                    acc0, out_hbm.at[pl.ds(row0, _SUB)], so0).wait()
            reduce_into(buf0, acc0)
            pltpu.async_copy(acc0, out_hbm.at[pl.ds(row0, _SUB)], so0)

            drain_gather(buf1, sg1)
            tcn = jnp.where(tc == n_chunks - 1, 0, tc + 1)
            comp_idx(tcn)
            fire_gather(buf0, sg0, 0)

            @pl.when(tc != 0)
            def _():
                pltpu.make_async_copy(
                    acc1, out_hbm.at[pl.ds(row0 + _SUB, _SUB)], so1).wait()
            reduce_into(buf1, acc1)
            pltpu.async_copy(acc1, out_hbm.at[pl.ds(row0 + _SUB, _SUB)], so1)
            return carry

        lax.fori_loop(0, n_chunks, body, 0)

        drain_gather(buf0, sg0)
        pltpu.make_async_copy(
            acc0, out_hbm.at[pl.ds(base, _SUB)], so0).wait()
        pltpu.make_async_copy(
            acc1, out_hbm.at[pl.ds(base + _SUB, _SUB)], so1).wait()

    return kern(ids, table)


_TB = 512


def _tc_dense(hidden, msum, w1b, w2, b1, b2, mergeb, bmerge, blk0, prev):
    n = hidden.shape[0]
    grid = (msum.shape[0] // _TB,)
    dn = (((1,), (1,)), ((), ()))

    def body(*refs):
        if prev is None:
            (hid_ref, sum_ref, w1_ref, w2_ref, b1_ref, b2_ref,
             wmg_ref, bm_ref, out_ref) = refs
        else:
            (_, hid_ref, sum_ref, w1_ref, w2_ref, b1_ref, b2_ref,
             wmg_ref, bm_ref, out_ref) = refs
        hid = hid_ref[...]
        rm = sum_ref[...] * 0.25
        rmb = rm.astype(jnp.bfloat16)
        pre = lax.dot_general(hid.astype(jnp.bfloat16), w1_ref[:, :_D], dn,
                              preferred_element_type=jnp.float32)
        pre += lax.dot_general(rmb, w1_ref[:, _D:], dn,
                               preferred_element_type=jnp.float32)
        mrg = lax.dot_general(rmb, wmg_ref[...], dn,
                              preferred_element_type=jnp.float32)
        pre += b1_ref[...]
        h = jnp.maximum(pre, 0.0)
        g = jnp.sum(h * w2_ref[...], axis=1, keepdims=True) + b2_ref[...]
        g = jax.nn.sigmoid(g)
        out_ref[...] = hid + g * mrg + bm_ref[...]

    in_specs = [
        pl.BlockSpec((_TB, _D), lambda i: (i + blk0, 0)),
        pl.BlockSpec((_TB, _D), lambda i: (i, 0)),
        pl.BlockSpec((_D, 2 * _D), lambda i: (0, 0)),
        pl.BlockSpec((1, _D), lambda i: (0, 0)),
        pl.BlockSpec((1, _D), lambda i: (0, 0)),
        pl.BlockSpec((1, 1), lambda i: (0, 0)),
        pl.BlockSpec((_D, _D), lambda i: (0, 0)),
        pl.BlockSpec((1, _D), lambda i: (0, 0)),
    ]
    args = [hidden, msum, w1b, w2, b1, b2, mergeb, bmerge]
    aliases = {}
    if prev is not None:
        in_specs = [pl.BlockSpec(memory_space=pl.ANY)] + in_specs
        args = [prev] + args
        aliases = {0: 0}
    return pl.pallas_call(
        body,
        grid=grid,
        in_specs=in_specs,
        out_specs=pl.BlockSpec((_TB, _D), lambda i: (i + blk0, 0)),
        out_shape=jax.ShapeDtypeStruct((n, _D), jnp.float32),
        input_output_aliases=aliases,
    )(*args)


def kernel(hidden_states, input_ids, memory_table, gate_w1, gate_b1,
           gate_w2, gate_b2, merge_w, merge_b):
    b, s, d = hidden_states.shape
    n = b * s
    ids = input_ids.reshape(n)
    hidden = hidden_states.reshape(n, d)

    w1b = gate_w1.astype(jnp.bfloat16)
    mergeb = merge_w.astype(jnp.bfloat16)
    b1 = gate_b1.reshape(1, d)
    b2 = gate_b2.reshape(1, 1)
    bm = merge_b.reshape(1, d)

    n_slices = 2
    ns = n // n_slices
    msums = [_sc_gather_sum(ids, memory_table, i * ns, ns)
             for i in range(n_slices)]
    out = None
    for i in range(n_slices):
        out = _tc_dense(hidden, msums[i], w1b, gate_w2, b1, b2,
                        mergeb, bm, i * (ns // _TB), out)
    return out.reshape(b, s, d)


# speedup vs baseline: 1.3076x; 1.3076x over previous
import functools

import jax
import jax.numpy as jnp
from jax import lax
from jax.experimental import pallas as pl
from jax.experimental.pallas import tpu as pltpu
from jax.experimental.pallas import tpu_sc as plsc

_TABLE = 100000
_D = 1024
_PRIMES = (17, 31, 53, 79)
_NC = 2
_NS = 16
_NW = _NC * _NS
_CHUNK = 16


_SUB = _CHUNK // 2


def _sc_gather_sum(ids, table, off, n_out):
    n_per_w = n_out // _NW
    n_chunks = n_per_w // _CHUNK
    mesh = plsc.VectorSubcoreMesh(
        core_axis_name="c", subcore_axis_name="s",
        num_cores=_NC, num_subcores=_NS,
    )

    @functools.partial(
        pl.kernel,
        out_type=jax.ShapeDtypeStruct((n_out, _D), jnp.float32),
        mesh=mesh,
        scratch_types=[
            pltpu.VMEM((n_per_w,), jnp.int32),
            pltpu.VMEM((4 * _CHUNK,), jnp.int32),
            pltpu.VMEM((4 * _SUB, _D), jnp.float32),
            pltpu.VMEM((4 * _SUB, _D), jnp.float32),
            pltpu.VMEM((_SUB, _D), jnp.float32),
            pltpu.VMEM((_SUB, _D), jnp.float32),
            pltpu.SemaphoreType.DMA,
            pltpu.SemaphoreType.DMA,
            pltpu.SemaphoreType.DMA,
            pltpu.SemaphoreType.DMA,
        ],
    )
    def kern(ids_hbm, table_hbm, out_hbm, ids_v, idx_v,
             buf0, buf1, acc0, acc1, sg0, sg1, so0, so1):
        wid = lax.axis_index("s") * _NC + lax.axis_index("c")
        base = wid * n_per_w
        pltpu.sync_copy(ids_hbm.at[pl.ds(off + base, n_per_w)], ids_v)

        def comp_idx(tc):
            ids16 = ids_v[pl.ds(tc * _CHUNK, _CHUNK)]
            for h in range(4):
                idx_v[pl.ds(h * _CHUNK, _CHUNK)] = (ids16 * _PRIMES[h]) % _TABLE

        def fire_gather(buf, sem, off):
            for h in range(4):
                pltpu.async_copy(
                    table_hbm.at[idx_v.at[pl.ds(h * _CHUNK + off, _SUB)]],
                    buf.at[pl.ds(h * _SUB, _SUB)], sem)

        def drain_gather(buf, sem):
            pltpu.make_async_copy(table_hbm.at[pl.ds(0, 4 * _SUB)], buf,
                                  sem).wait()

        def reduce_into(buf, acc):
            def rj(j, c):
                s = pl.ds(j * 16, 16)
                for t in range(_SUB):
                    acc[t, s] = ((buf[t, s] + buf[_SUB + t, s])
                                 + (buf[2 * _SUB + t, s]
                                    + buf[3 * _SUB + t, s]))
                return c
            lax.fori_loop(0, _D // 16, rj, 0)

        comp_idx(0)
        fire_gather(buf0, sg0, 0)

        def body(tc, carry):
            row0 = base + tc * _CHUNK
            fire_gather(buf1, sg1, _SUB)

            drain_gather(buf0, sg0)

            @pl.when(tc != 0)
            def _():
                pltpu.make_async_copy(
                    acc0, out_hbm.at[pl.ds(row0, _SUB)], so0).wait()
            reduce_into(buf0, acc0)
            pltpu.async_copy(acc0, out_hbm.at[pl.ds(row0, _SUB)], so0)

            drain_gather(buf1, sg1)
            tcn = jnp.where(tc == n_chunks - 1, 0, tc + 1)
            comp_idx(tcn)
            fire_gather(buf0, sg0, 0)

            @pl.when(tc != 0)
            def _():
                pltpu.make_async_copy(
                    acc1, out_hbm.at[pl.ds(row0 + _SUB, _SUB)], so1).wait()
            reduce_into(buf1, acc1)
            pltpu.async_copy(acc1, out_hbm.at[pl.ds(row0 + _SUB, _SUB)], so1)
            return carry

        lax.fori_loop(0, n_chunks, body, 0)

        drain_gather(buf0, sg0)
        pltpu.make_async_copy(
            acc0, out_hbm.at[pl.ds(base, _SUB)], so0).wait()
        pltpu.make_async_copy(
            acc1, out_hbm.at[pl.ds(base + _SUB, _SUB)], so1).wait()

    return kern(ids, table)


_TB = 512


def _tc_dense(hidden, msum, w1b, w2, b1, b2, mergeb, bmerge, blk0, prev):
    n = hidden.shape[0]
    grid = (msum.shape[0] // _TB,)
    dn = (((1,), (1,)), ((), ()))

    def body(*refs):
        if prev is None:
            (hid_ref, sum_ref, w1_ref, w2_ref, b1_ref, b2_ref,
             wmg_ref, bm_ref, out_ref) = refs
        else:
            (_, hid_ref, sum_ref, w1_ref, w2_ref, b1_ref, b2_ref,
             wmg_ref, bm_ref, out_ref) = refs
        hid = hid_ref[...]
        rm = sum_ref[...] * 0.25
        rmb = rm.astype(jnp.bfloat16)
        pre = lax.dot_general(hid.astype(jnp.bfloat16), w1_ref[:, :_D], dn,
                              preferred_element_type=jnp.float32)
        pre += lax.dot_general(rmb, w1_ref[:, _D:], dn,
                               preferred_element_type=jnp.float32)
        mrg = lax.dot_general(rmb, wmg_ref[...], dn,
                              preferred_element_type=jnp.float32)
        pre += b1_ref[...]
        h = jnp.maximum(pre, 0.0)
        g = jnp.sum(h * w2_ref[...], axis=1, keepdims=True) + b2_ref[...]
        g = jax.nn.sigmoid(g)
        out_ref[...] = hid + g * mrg + bm_ref[...]

    in_specs = [
        pl.BlockSpec((_TB, _D), lambda i: (i + blk0, 0)),
        pl.BlockSpec((_TB, _D), lambda i: (i, 0)),
        pl.BlockSpec((_D, 2 * _D), lambda i: (0, 0)),
        pl.BlockSpec((1, _D), lambda i: (0, 0)),
        pl.BlockSpec((1, _D), lambda i: (0, 0)),
        pl.BlockSpec((1, 1), lambda i: (0, 0)),
        pl.BlockSpec((_D, _D), lambda i: (0, 0)),
        pl.BlockSpec((1, _D), lambda i: (0, 0)),
    ]
    args = [hidden, msum, w1b, w2, b1, b2, mergeb, bmerge]
    aliases = {}
    if prev is not None:
        in_specs = [pl.BlockSpec(memory_space=pl.ANY)] + in_specs
        args = [prev] + args
        aliases = {0: 0}
    return pl.pallas_call(
        body,
        grid=grid,
        in_specs=in_specs,
        out_specs=pl.BlockSpec((_TB, _D), lambda i: (i + blk0, 0)),
        out_shape=jax.ShapeDtypeStruct((n, _D), jnp.float32),
        input_output_aliases=aliases,
    )(*args)


def kernel(hidden_states, input_ids, memory_table, gate_w1, gate_b1,
           gate_w2, gate_b2, merge_w, merge_b):
    b, s, d = hidden_states.shape
    n = b * s
    ids = input_ids.reshape(n)
    hidden = hidden_states.reshape(n, d)

    w1b = gate_w1.astype(jnp.bfloat16)
    mergeb = merge_w.astype(jnp.bfloat16)
    b1 = gate_b1.reshape(1, d)
    b2 = gate_b2.reshape(1, 1)
    bm = merge_b.reshape(1, d)

    n_slices = 2
    ns = n // n_slices
    msums = [_sc_gather_sum(ids, memory_table, i * ns, ns)
             for i in range(n_slices)]
    out = None
    for i in range(n_slices):
        out = _tc_dense(hidden, msums[i], w1b, gate_w2, b1, b2,
                        mergeb, bm, i * (ns // _TB), out)
    return out.reshape(b, s, d)


# speedup vs baseline: 1.3505x; 1.0328x over previous
import functools

import jax
import jax.numpy as jnp
from jax import lax
from jax.experimental import pallas as pl
from jax.experimental.pallas import tpu as pltpu
from jax.experimental.pallas import tpu_sc as plsc

_TABLE = 100000
_D = 1024
_PRIMES = (17, 31, 53, 79)
_NC = 2
_NS = 16
_NW = _NC * _NS
_CHUNK = 16


_SUB = _CHUNK // 2


def _sc_gather_sum(ids, table, off, n_out):
    n_per_w = n_out // _NW
    n_chunks = n_per_w // _CHUNK
    mesh = plsc.VectorSubcoreMesh(
        core_axis_name="c", subcore_axis_name="s",
        num_cores=_NC, num_subcores=_NS,
    )

    @functools.partial(
        pl.kernel,
        out_type=jax.ShapeDtypeStruct((n_out, _D), jnp.float32),
        mesh=mesh,
        scratch_types=[
            pltpu.VMEM((n_per_w,), jnp.int32),
            pltpu.VMEM((4 * _CHUNK,), jnp.int32),
            pltpu.VMEM((4 * _SUB, _D), jnp.float32),
            pltpu.VMEM((4 * _SUB, _D), jnp.float32),
            pltpu.VMEM((_SUB, _D), jnp.float32),
            pltpu.VMEM((_SUB, _D), jnp.float32),
            pltpu.SemaphoreType.DMA,
            pltpu.SemaphoreType.DMA,
            pltpu.SemaphoreType.DMA,
            pltpu.SemaphoreType.DMA,
        ],
    )
    def kern(ids_hbm, table_hbm, out_hbm, ids_v, idx_v,
             buf0, buf1, acc0, acc1, sg0, sg1, so0, so1):
        wid = lax.axis_index("s") * _NC + lax.axis_index("c")
        base = wid * n_per_w
        pltpu.sync_copy(ids_hbm.at[pl.ds(off + base, n_per_w)], ids_v)

        def comp_idx(tc):
            ids16 = ids_v[pl.ds(tc * _CHUNK, _CHUNK)]
            for h in range(4):
                idx_v[pl.ds(h * _CHUNK, _CHUNK)] = (ids16 * _PRIMES[h]) % _TABLE

        def fire_gather(buf, sem, off):
            for h in range(4):
                pltpu.async_copy(
                    table_hbm.at[idx_v.at[pl.ds(h * _CHUNK + off, _SUB)]],
                    buf.at[pl.ds(h * _SUB, _SUB)], sem)

        def drain_gather(buf, sem):
            pltpu.make_async_copy(table_hbm.at[pl.ds(0, 4 * _SUB)], buf,
                                  sem).wait()

        def reduce_into(buf, acc):
            def rj(j, c):
                s = pl.ds(j * 16, 16)
                for t in range(_SUB):
                    acc[t, s] = ((buf[t, s] + buf[_SUB + t, s])
                                 + (buf[2 * _SUB + t, s]
                                    + buf[3 * _SUB + t, s]))
                return c
            lax.fori_loop(0, _D // 16, rj, 0)

        comp_idx(0)
        fire_gather(buf0, sg0, 0)

        def body(tc, carry):
            row0 = base + tc * _CHUNK
            fire_gather(buf1, sg1, _SUB)

            drain_gather(buf0, sg0)

            @pl.when(tc != 0)
            def _():
                pltpu.make_async_copy(
                    acc0, out_hbm.at[pl.ds(row0, _SUB)], so0).wait()
            reduce_into(buf0, acc0)
            pltpu.async_copy(acc0, out_hbm.at[pl.ds(row0, _SUB)], so0)

            drain_gather(buf1, sg1)
            tcn = jnp.where(tc == n_chunks - 1, 0, tc + 1)
            comp_idx(tcn)
            fire_gather(buf0, sg0, 0)

            @pl.when(tc != 0)
            def _():
                pltpu.make_async_copy(
                    acc1, out_hbm.at[pl.ds(row0 + _SUB, _SUB)], so1).wait()
            reduce_into(buf1, acc1)
            pltpu.async_copy(acc1, out_hbm.at[pl.ds(row0 + _SUB, _SUB)], so1)
            return carry

        lax.fori_loop(0, n_chunks, body, 0)

        drain_gather(buf0, sg0)
        pltpu.make_async_copy(
            acc0, out_hbm.at[pl.ds(base, _SUB)], so0).wait()
        pltpu.make_async_copy(
            acc1, out_hbm.at[pl.ds(base + _SUB, _SUB)], so1).wait()

    return kern(ids, table)


_TB = 512


def _tc_dense(hidden, msum, w1b, w2, b1, b2, mergeb, bmerge, blk0, prev):
    n = hidden.shape[0]
    grid = (msum.shape[0] // _TB,)
    dn = (((1,), (1,)), ((), ()))

    def body(*refs):
        if prev is None:
            (hid_ref, sum_ref, w1_ref, w2_ref, b1_ref, b2_ref,
             wmg_ref, bm_ref, out_ref) = refs
        else:
            (_, hid_ref, sum_ref, w1_ref, w2_ref, b1_ref, b2_ref,
             wmg_ref, bm_ref, out_ref) = refs
        hid = hid_ref[...]
        rm = sum_ref[...] * 0.25
        rmb = rm.astype(jnp.bfloat16)
        pre = lax.dot_general(hid.astype(jnp.bfloat16), w1_ref[:, :_D], dn,
                              preferred_element_type=jnp.float32)
        pre += lax.dot_general(rmb, w1_ref[:, _D:], dn,
                               preferred_element_type=jnp.float32)
        mrg = lax.dot_general(rmb, wmg_ref[...], dn,
                              preferred_element_type=jnp.float32)
        pre += b1_ref[...]
        h = jnp.maximum(pre, 0.0)
        g = jnp.sum(h * w2_ref[...], axis=1, keepdims=True) + b2_ref[...]
        g = jax.nn.sigmoid(g)
        out_ref[...] = hid + g * mrg + bm_ref[...]

    in_specs = [
        pl.BlockSpec((_TB, _D), lambda i: (i + blk0, 0)),
        pl.BlockSpec((_TB, _D), lambda i: (i, 0)),
        pl.BlockSpec((_D, 2 * _D), lambda i: (0, 0)),
        pl.BlockSpec((1, _D), lambda i: (0, 0)),
        pl.BlockSpec((1, _D), lambda i: (0, 0)),
        pl.BlockSpec((1, 1), lambda i: (0, 0)),
        pl.BlockSpec((_D, _D), lambda i: (0, 0)),
        pl.BlockSpec((1, _D), lambda i: (0, 0)),
    ]
    args = [hidden, msum, w1b, w2, b1, b2, mergeb, bmerge]
    aliases = {}
    if prev is not None:
        in_specs = [pl.BlockSpec(memory_space=pl.ANY)] + in_specs
        args = [prev] + args
        aliases = {0: 0}
    return pl.pallas_call(
        body,
        grid=grid,
        in_specs=in_specs,
        out_specs=pl.BlockSpec((_TB, _D), lambda i: (i + blk0, 0)),
        out_shape=jax.ShapeDtypeStruct((n, _D), jnp.float32),
        input_output_aliases=aliases,
    )(*args)


def kernel(hidden_states, input_ids, memory_table, gate_w1, gate_b1,
           gate_w2, gate_b2, merge_w, merge_b):
    b, s, d = hidden_states.shape
    n = b * s
    ids = input_ids.reshape(n)
    hidden = hidden_states.reshape(n, d)

    w1b = gate_w1.astype(jnp.bfloat16)
    mergeb = merge_w.astype(jnp.bfloat16)
    b1 = gate_b1.reshape(1, d)
    b2 = gate_b2.reshape(1, 1)
    bm = merge_b.reshape(1, d)

    slices = [5 * n // 8, 3 * n // 8]
    offs = [0, 5 * n // 8]
    msums = [_sc_gather_sum(ids, memory_table, offs[i], slices[i])
             for i in range(len(slices))]
    out = None
    for i in range(len(slices)):
        out = _tc_dense(hidden, msums[i], w1b, gate_w2, b1, b2,
                        mergeb, bm, offs[i] // _TB, out)
    return out.reshape(b, s, d)
